# Initial kernel scaffold; baseline (speedup 1.0000x reference)
import jax
import jax.numpy as jnp
from jax.experimental import pallas as pl


def kernel(x, A_hat, embed, W11, b11, W12, b12, W21, b21, W22, b22, W31, b31, W32, b32):
    raise NotImplementedError("write your pallas kernel here")


# speedup vs baseline: 1.0123x; 1.0123x over previous
import functools

import jax
import jax.numpy as jnp
from jax.experimental import pallas as pl


def _layer_body(a_ref, xf_ref, xb_ref, w1_ref, b1_ref, w2_ref, b2_ref, o_ref):
    y = jnp.dot(a_ref[...], xf_ref[...], preferred_element_type=jnp.float32)
    xb = xb_ref[...]
    s = y + xb
    p = y * xb
    t = jax.lax.dot_general(
        s, w1_ref[...], (((1,), (1,)), ((), ())),
        preferred_element_type=jnp.float32)
    t = t + jax.lax.dot_general(
        p, w2_ref[...], (((1,), (1,)), ((), ())),
        preferred_element_type=jnp.float32)
    t = t + b1_ref[...] + b2_ref[...]
    o_ref[...] = jnp.where(t >= 0, t, 0.2 * t)


@functools.partial(jax.jit, static_argnames=("bm",))
def _layer(a_hat, x_prev, w1, b1, w2, b2, bm=400):
    n, d = x_prev.shape
    m = a_hat.shape[0]
    grid = (pl.cdiv(m, bm),)
    return pl.pallas_call(
        _layer_body,
        grid=grid,
        in_specs=[
            pl.BlockSpec((bm, n), lambda i: (i, 0)),
            pl.BlockSpec((n, d), lambda i: (0, 0)),
            pl.BlockSpec((bm, d), lambda i: (i, 0)),
            pl.BlockSpec((d, d), lambda i: (0, 0)),
            pl.BlockSpec((1, d), lambda i: (0, 0)),
            pl.BlockSpec((d, d), lambda i: (0, 0)),
            pl.BlockSpec((1, d), lambda i: (0, 0)),
        ],
        out_specs=pl.BlockSpec((bm, d), lambda i: (i, 0)),
        out_shape=jax.ShapeDtypeStruct((m, d), jnp.float32),
    )(a_hat, x_prev, x_prev, w1, b1, w2, b2)


def kernel(x, A_hat, embed, W11, b11, W12, b12, W21, b21, W22, b22, W31, b31,
           W32, b32):
    x0 = jnp.take(embed, x, axis=0)
    b = [b.reshape(1, -1) for b in (b11, b12, b21, b22, b31, b32)]
    x1 = _layer(A_hat, x0, W11, b[0], W12, b[1])
    x2 = _layer(A_hat, x1, W21, b[2], W22, b[3])
    x3 = _layer(A_hat, x2, W31, b[4], W32, b[5])
    return jnp.concatenate((x0, x1, x2, x3), axis=1)


# speedup vs baseline: 1.0764x; 1.0633x over previous
import functools

import jax
import jax.numpy as jnp
from jax.experimental import pallas as pl


def _epilogue(y, xb, w1_ref, b1_ref, w2_ref, b2_ref, o_ref):
    s = y + xb
    p = y * xb
    t = jax.lax.dot_general(
        s, w1_ref[...], (((1,), (1,)), ((), ())),
        preferred_element_type=jnp.float32)
    t = t + jax.lax.dot_general(
        p, w2_ref[...], (((1,), (1,)), ((), ())),
        preferred_element_type=jnp.float32)
    t = t + b1_ref[...] + b2_ref[...]
    o_ref[...] = jnp.where(t >= 0, t, 0.2 * t)


def _layer1_body(a_ref, xf_ref, xb_ref, w1_ref, b1_ref, w2_ref, b2_ref,
                 o_ref, a16_ref):
    a = a_ref[...]
    a16_ref[...] = a.astype(jnp.bfloat16)
    y = jnp.dot(a, xf_ref[...], preferred_element_type=jnp.float32)
    _epilogue(y, xb_ref[...], w1_ref, b1_ref, w2_ref, b2_ref, o_ref)


def _layer_bf16_body(a_ref, xf_ref, xb_ref, w1_ref, b1_ref, w2_ref, b2_ref,
                     o_ref):
    y = jnp.dot(a_ref[...], xf_ref[...], preferred_element_type=jnp.float32)
    _epilogue(y, xb_ref[...], w1_ref, b1_ref, w2_ref, b2_ref, o_ref)


@functools.partial(jax.jit, static_argnames=("bm",))
def _layer1(a_hat, x_prev, w1, b1, w2, b2, bm=200):
    n, d = x_prev.shape
    m = a_hat.shape[0]
    return pl.pallas_call(
        _layer1_body,
        grid=(pl.cdiv(m, bm),),
        in_specs=[
            pl.BlockSpec((bm, n), lambda i: (i, 0)),
            pl.BlockSpec((n, d), lambda i: (0, 0)),
            pl.BlockSpec((bm, d), lambda i: (i, 0)),
            pl.BlockSpec((d, d), lambda i: (0, 0)),
            pl.BlockSpec((1, d), lambda i: (0, 0)),
            pl.BlockSpec((d, d), lambda i: (0, 0)),
            pl.BlockSpec((1, d), lambda i: (0, 0)),
        ],
        out_specs=[
            pl.BlockSpec((bm, d), lambda i: (i, 0)),
            pl.BlockSpec((bm, n), lambda i: (i, 0)),
        ],
        out_shape=[
            jax.ShapeDtypeStruct((m, d), jnp.float32),
            jax.ShapeDtypeStruct((m, n), jnp.bfloat16),
        ],
    )(a_hat, x_prev, x_prev, w1, b1, w2, b2)


@functools.partial(jax.jit, static_argnames=("bm",))
def _layer_bf16(a16, x_prev, w1, b1, w2, b2, bm=400):
    n, d = x_prev.shape
    m = a16.shape[0]
    x16 = x_prev.astype(jnp.bfloat16)
    return pl.pallas_call(
        _layer_bf16_body,
        grid=(pl.cdiv(m, bm),),
        in_specs=[
            pl.BlockSpec((bm, n), lambda i: (i, 0)),
            pl.BlockSpec((n, d), lambda i: (0, 0)),
            pl.BlockSpec((bm, d), lambda i: (i, 0)),
            pl.BlockSpec((d, d), lambda i: (0, 0)),
            pl.BlockSpec((1, d), lambda i: (0, 0)),
            pl.BlockSpec((d, d), lambda i: (0, 0)),
            pl.BlockSpec((1, d), lambda i: (0, 0)),
        ],
        out_specs=pl.BlockSpec((bm, d), lambda i: (i, 0)),
        out_shape=jax.ShapeDtypeStruct((m, d), jnp.float32),
    )(a16, x16, x_prev, w1, b1, w2, b2)


def kernel(x, A_hat, embed, W11, b11, W12, b12, W21, b21, W22, b22, W31, b31,
           W32, b32):
    x0 = jnp.take(embed, x, axis=0)
    b = [b.reshape(1, -1) for b in (b11, b12, b21, b22, b31, b32)]
    x1, a16 = _layer1(A_hat, x0, W11, b[0], W12, b[1])
    x2 = _layer_bf16(a16, x1, W21, b[2], W22, b[3])
    x3 = _layer_bf16(a16, x2, W31, b[4], W32, b[5])
    return jnp.concatenate((x0, x1, x2, x3), axis=1)


# speedup vs baseline: 1.0917x; 1.0142x over previous
import functools

import jax
import jax.numpy as jnp
from jax.experimental import pallas as pl


def _epilogue(y, xb, w1_ref, b1_ref, w2_ref, b2_ref, o_ref, o16_ref):
    s = y + xb
    p = y * xb
    t = jax.lax.dot_general(
        s, w1_ref[...], (((1,), (1,)), ((), ())),
        preferred_element_type=jnp.float32)
    t = t + jax.lax.dot_general(
        p, w2_ref[...], (((1,), (1,)), ((), ())),
        preferred_element_type=jnp.float32)
    t = t + b1_ref[...] + b2_ref[...]
    t = jnp.where(t >= 0, t, 0.2 * t)
    o_ref[...] = t
    o16_ref[...] = t.astype(jnp.bfloat16)


def _layer1_body(a_ref, xf_ref, xb_ref, w1_ref, b1_ref, w2_ref, b2_ref,
                 o_ref, o16_ref, a16_ref):
    a16 = a_ref[...].astype(jnp.bfloat16)
    a16_ref[...] = a16
    y = jnp.dot(a16, xf_ref[...], preferred_element_type=jnp.float32)
    _epilogue(y, xb_ref[...], w1_ref, b1_ref, w2_ref, b2_ref, o_ref, o16_ref)


def _layer_bf16_body(a_ref, xf_ref, xb_ref, w1_ref, b1_ref, w2_ref, b2_ref,
                     o_ref, o16_ref):
    y = jnp.dot(a_ref[...], xf_ref[...], preferred_element_type=jnp.float32)
    _epilogue(y, xb_ref[...], w1_ref, b1_ref, w2_ref, b2_ref, o_ref, o16_ref)


@functools.partial(jax.jit, static_argnames=("bm",))
def _layer1(a_hat, x16, x_prev, w1, b1, w2, b2, bm=200):
    n, d = x_prev.shape
    m = a_hat.shape[0]
    return pl.pallas_call(
        _layer1_body,
        grid=(pl.cdiv(m, bm),),
        in_specs=[
            pl.BlockSpec((bm, n), lambda i: (i, 0)),
            pl.BlockSpec((n, d), lambda i: (0, 0)),
            pl.BlockSpec((bm, d), lambda i: (i, 0)),
            pl.BlockSpec((d, d), lambda i: (0, 0)),
            pl.BlockSpec((1, d), lambda i: (0, 0)),
            pl.BlockSpec((d, d), lambda i: (0, 0)),
            pl.BlockSpec((1, d), lambda i: (0, 0)),
        ],
        out_specs=[
            pl.BlockSpec((bm, d), lambda i: (i, 0)),
            pl.BlockSpec((bm, d), lambda i: (i, 0)),
            pl.BlockSpec((bm, n), lambda i: (i, 0)),
        ],
        out_shape=[
            jax.ShapeDtypeStruct((m, d), jnp.float32),
            jax.ShapeDtypeStruct((m, d), jnp.bfloat16),
            jax.ShapeDtypeStruct((m, n), jnp.bfloat16),
        ],
    )(a_hat, x16, x_prev, w1, b1, w2, b2)


@functools.partial(jax.jit, static_argnames=("bm",))
def _layer_bf16(a16, x16, x_prev, w1, b1, w2, b2, bm=1000):
    n, d = x_prev.shape
    m = a16.shape[0]
    return pl.pallas_call(
        _layer_bf16_body,
        grid=(pl.cdiv(m, bm),),
        in_specs=[
            pl.BlockSpec((bm, n), lambda i: (i, 0)),
            pl.BlockSpec((n, d), lambda i: (0, 0)),
            pl.BlockSpec((bm, d), lambda i: (i, 0)),
            pl.BlockSpec((d, d), lambda i: (0, 0)),
            pl.BlockSpec((1, d), lambda i: (0, 0)),
            pl.BlockSpec((d, d), lambda i: (0, 0)),
            pl.BlockSpec((1, d), lambda i: (0, 0)),
        ],
        out_specs=[
            pl.BlockSpec((bm, d), lambda i: (i, 0)),
            pl.BlockSpec((bm, d), lambda i: (i, 0)),
        ],
        out_shape=[
            jax.ShapeDtypeStruct((m, d), jnp.float32),
            jax.ShapeDtypeStruct((m, d), jnp.bfloat16),
        ],
    )(a16, x16, x_prev, w1, b1, w2, b2)


def kernel(x, A_hat, embed, W11, b11, W12, b12, W21, b21, W22, b22, W31, b31,
           W32, b32):
    x0 = jnp.take(embed, x, axis=0)
    x0_16 = x0.astype(jnp.bfloat16)
    b = [b.reshape(1, -1) for b in (b11, b12, b21, b22, b31, b32)]
    x1, x1_16, a16 = _layer1(A_hat, x0_16, x0, W11, b[0], W12, b[1])
    x2, x2_16 = _layer_bf16(a16, x1_16, x1, W21, b[2], W22, b[3])
    x3, _ = _layer_bf16(a16, x2_16, x2, W31, b[4], W32, b[5])
    return jnp.concatenate((x0, x1, x2, x3), axis=1)


# speedup vs baseline: 1.1131x; 1.0196x over previous
import functools

import jax
import jax.numpy as jnp
from jax.experimental import pallas as pl


def _epilogue(y, xb, w1_ref, b1_ref, w2_ref, b2_ref, o_ref, o16_ref):
    s = y + xb
    p = y * xb
    t = jax.lax.dot_general(
        s, w1_ref[...], (((1,), (1,)), ((), ())),
        preferred_element_type=jnp.float32)
    t = t + jax.lax.dot_general(
        p, w2_ref[...], (((1,), (1,)), ((), ())),
        preferred_element_type=jnp.float32)
    t = t + b1_ref[...] + b2_ref[...]
    t = jnp.where(t >= 0, t, 0.2 * t)
    o_ref[...] = t
    o16_ref[...] = t.astype(jnp.bfloat16)


def _layer1_body(a_ref, xf_ref, xb_ref, w1_ref, b1_ref, w2_ref, b2_ref,
                 o_ref, o16_ref, a16_ref):
    a16 = a_ref[...].astype(jnp.bfloat16)
    a16_ref[...] = a16
    y = jnp.dot(a16, xf_ref[...], preferred_element_type=jnp.float32)
    _epilogue(y, xb_ref[...], w1_ref, b1_ref, w2_ref, b2_ref, o_ref, o16_ref)


def _layer_bf16_body(a_ref, xf_ref, xb_ref, w1_ref, b1_ref, w2_ref, b2_ref,
                     o_ref, o16_ref):
    y = jnp.dot(a_ref[...], xf_ref[...], preferred_element_type=jnp.float32)
    _epilogue(y, xb_ref[...], w1_ref, b1_ref, w2_ref, b2_ref, o_ref, o16_ref)


@functools.partial(jax.jit, static_argnames=("bm",))
def _layer1(a_hat, x16, x_prev, w1, b1, w2, b2, bm=320):
    n, d = x_prev.shape
    m = a_hat.shape[0]
    return pl.pallas_call(
        _layer1_body,
        grid=(pl.cdiv(m, bm),),
        in_specs=[
            pl.BlockSpec((bm, n), lambda i: (i, 0)),
            pl.BlockSpec((n, d), lambda i: (0, 0)),
            pl.BlockSpec((bm, d), lambda i: (i, 0)),
            pl.BlockSpec((d, d), lambda i: (0, 0)),
            pl.BlockSpec((1, d), lambda i: (0, 0)),
            pl.BlockSpec((d, d), lambda i: (0, 0)),
            pl.BlockSpec((1, d), lambda i: (0, 0)),
        ],
        out_specs=[
            pl.BlockSpec((bm, d), lambda i: (i, 0)),
            pl.BlockSpec((bm, d), lambda i: (i, 0)),
            pl.BlockSpec((bm, n), lambda i: (i, 0)),
        ],
        out_shape=[
            jax.ShapeDtypeStruct((m, d), jnp.float32),
            jax.ShapeDtypeStruct((m, d), jnp.bfloat16),
            jax.ShapeDtypeStruct((m, n), jnp.bfloat16),
        ],
    )(a_hat, x16, x_prev, w1, b1, w2, b2)


@functools.partial(jax.jit, static_argnames=("bm",))
def _layer_bf16(a16, x16, x_prev, w1, b1, w2, b2, bm=1000):
    n, d = x_prev.shape
    m = a16.shape[0]
    return pl.pallas_call(
        _layer_bf16_body,
        grid=(pl.cdiv(m, bm),),
        in_specs=[
            pl.BlockSpec((bm, n), lambda i: (i, 0)),
            pl.BlockSpec((n, d), lambda i: (0, 0)),
            pl.BlockSpec((bm, d), lambda i: (i, 0)),
            pl.BlockSpec((d, d), lambda i: (0, 0)),
            pl.BlockSpec((1, d), lambda i: (0, 0)),
            pl.BlockSpec((d, d), lambda i: (0, 0)),
            pl.BlockSpec((1, d), lambda i: (0, 0)),
        ],
        out_specs=[
            pl.BlockSpec((bm, d), lambda i: (i, 0)),
            pl.BlockSpec((bm, d), lambda i: (i, 0)),
        ],
        out_shape=[
            jax.ShapeDtypeStruct((m, d), jnp.float32),
            jax.ShapeDtypeStruct((m, d), jnp.bfloat16),
        ],
    )(a16, x16, x_prev, w1, b1, w2, b2)


def kernel(x, A_hat, embed, W11, b11, W12, b12, W21, b21, W22, b22, W31, b31,
           W32, b32):
    x0 = jnp.take(embed, x, axis=0)
    x0_16 = x0.astype(jnp.bfloat16)
    b = [b.reshape(1, -1) for b in (b11, b12, b21, b22, b31, b32)]
    x1, x1_16, a16 = _layer1(A_hat, x0_16, x0, W11, b[0], W12, b[1])
    x2, x2_16 = _layer_bf16(a16, x1_16, x1, W21, b[2], W22, b[3])
    x3, _ = _layer_bf16(a16, x2_16, x2, W31, b[4], W32, b[5])
    return jnp.concatenate((x0, x1, x2, x3), axis=1)


# speedup vs baseline: 1.1187x; 1.0051x over previous
import functools

import jax
import jax.numpy as jnp
from jax.experimental import pallas as pl


def _epilogue(y, xb, w1_ref, b1_ref, w2_ref, b2_ref, o_ref, o16_ref):
    s = y + xb
    p = y * xb
    t = jax.lax.dot_general(
        s, w1_ref[...], (((1,), (1,)), ((), ())),
        preferred_element_type=jnp.float32)
    t = t + jax.lax.dot_general(
        p, w2_ref[...], (((1,), (1,)), ((), ())),
        preferred_element_type=jnp.float32)
    t = t + b1_ref[...] + b2_ref[...]
    t = jnp.where(t >= 0, t, 0.2 * t)
    o_ref[...] = t
    o16_ref[...] = t.astype(jnp.bfloat16)


def _layer1_body(a_ref, xf_ref, xb_ref, w1_ref, b1_ref, w2_ref, b2_ref,
                 o_ref, o16_ref, a16_ref):
    a16 = a_ref[...].astype(jnp.bfloat16)
    a16_ref[...] = a16
    y = jnp.dot(a16, xf_ref[...], preferred_element_type=jnp.float32)
    _epilogue(y, xb_ref[...], w1_ref, b1_ref, w2_ref, b2_ref, o_ref, o16_ref)


def _layer_bf16_body(a_ref, xf_ref, xb_ref, w1_ref, b1_ref, w2_ref, b2_ref,
                     o_ref, o16_ref):
    y = jnp.dot(a_ref[...], xf_ref[...], preferred_element_type=jnp.float32)
    _epilogue(y, xb_ref[...], w1_ref, b1_ref, w2_ref, b2_ref, o_ref, o16_ref)


@functools.partial(jax.jit, static_argnames=("bm",))
def _layer1(a_hat, x16, x_prev, w1, b1, w2, b2, bm=400):
    n, d = x_prev.shape
    m = a_hat.shape[0]
    return pl.pallas_call(
        _layer1_body,
        grid=(pl.cdiv(m, bm),),
        in_specs=[
            pl.BlockSpec((bm, n), lambda i: (i, 0)),
            pl.BlockSpec((n, d), lambda i: (0, 0)),
            pl.BlockSpec((bm, d), lambda i: (i, 0)),
            pl.BlockSpec((d, d), lambda i: (0, 0)),
            pl.BlockSpec((1, d), lambda i: (0, 0)),
            pl.BlockSpec((d, d), lambda i: (0, 0)),
            pl.BlockSpec((1, d), lambda i: (0, 0)),
        ],
        out_specs=[
            pl.BlockSpec((bm, d), lambda i: (i, 0)),
            pl.BlockSpec((bm, d), lambda i: (i, 0)),
            pl.BlockSpec((bm, n), lambda i: (i, 0)),
        ],
        out_shape=[
            jax.ShapeDtypeStruct((m, d), jnp.float32),
            jax.ShapeDtypeStruct((m, d), jnp.bfloat16),
            jax.ShapeDtypeStruct((m, n), jnp.bfloat16),
        ],
    )(a_hat, x16, x_prev, w1, b1, w2, b2)


@functools.partial(jax.jit, static_argnames=("bm",))
def _layer_bf16(a16, x16, x_prev, w1, b1, w2, b2, bm=1000):
    n, d = x_prev.shape
    m = a16.shape[0]
    return pl.pallas_call(
        _layer_bf16_body,
        grid=(pl.cdiv(m, bm),),
        in_specs=[
            pl.BlockSpec((bm, n), lambda i: (i, 0)),
            pl.BlockSpec((n, d), lambda i: (0, 0)),
            pl.BlockSpec((bm, d), lambda i: (i, 0)),
            pl.BlockSpec((d, d), lambda i: (0, 0)),
            pl.BlockSpec((1, d), lambda i: (0, 0)),
            pl.BlockSpec((d, d), lambda i: (0, 0)),
            pl.BlockSpec((1, d), lambda i: (0, 0)),
        ],
        out_specs=[
            pl.BlockSpec((bm, d), lambda i: (i, 0)),
            pl.BlockSpec((bm, d), lambda i: (i, 0)),
        ],
        out_shape=[
            jax.ShapeDtypeStruct((m, d), jnp.float32),
            jax.ShapeDtypeStruct((m, d), jnp.bfloat16),
        ],
    )(a16, x16, x_prev, w1, b1, w2, b2)


def kernel(x, A_hat, embed, W11, b11, W12, b12, W21, b21, W22, b22, W31, b31,
           W32, b32):
    x0 = jnp.take(embed, x, axis=0)
    x0_16 = x0.astype(jnp.bfloat16)
    b = [b.reshape(1, -1) for b in (b11, b12, b21, b22, b31, b32)]
    x1, x1_16, a16 = _layer1(A_hat, x0_16, x0, W11, b[0], W12, b[1])
    x2, x2_16 = _layer_bf16(a16, x1_16, x1, W21, b[2], W22, b[3])
    x3, _ = _layer_bf16(a16, x2_16, x2, W31, b[4], W32, b[5])
    return jnp.concatenate((x0, x1, x2, x3), axis=1)
